# baseline (speedup 1.0000x reference)
import jax
import jax.numpy as jnp
from jax import lax
from jax.experimental import pallas as pl
from jax.experimental.pallas import tpu as pltpu
from jax.experimental.pallas import tpu_sc as plsc

_GDN = lax.GatherDimensionNumbers(
    offset_dims=(), collapsed_slice_dims=(0,), start_index_map=(0,))


def _shuf(t, idx):
    return lax.gather(t, idx[:, None], _GDN, (1,),
                      mode=lax.GatherScatterMode.PROMISE_IN_BOUNDS)


def _bfr(v):
    i = lax.bitcast_convert_type(v, jnp.int32)
    r = (i + 0x7FFF + ((i >> 16) & 1)) & jnp.int32(-65536)
    return lax.bitcast_convert_type(r, jnp.float32)


N = 10000
E = 320000
DF = 128
H = 64
DE = 16
G = 64

NC = 2
NS = 16
NW = NC * NS
EPW = E // NW
C = 80
NCHUNK = EPW // C
NPAD = 10240
RPS = NPAD // NS

W1 = 96
W2 = 80


def _node_xform_body(x_ref, wl_ref, bl_ref, wr_ref, br_ref, xl_ref, xr_ref):
    x = x_ref[...]
    xl_ref[...] = lax.dot_general(x, wl_ref[...], (((1,), (1,)), ((), ()))) + bl_ref[...]
    xr_ref[...] = lax.dot_general(x, wr_ref[...], (((1,), (1,)), ((), ()))) + br_ref[...]


def _node_xform(x, Wl, bl, Wr, br):
    BN = 1000
    return pl.pallas_call(
        _node_xform_body,
        grid=(N // BN,),
        in_specs=[
            pl.BlockSpec((BN, DF), lambda i: (i, 0)),
            pl.BlockSpec((H, DF), lambda i: (0, 0)),
            pl.BlockSpec((1, H), lambda i: (0, 0)),
            pl.BlockSpec((H, DF), lambda i: (0, 0)),
            pl.BlockSpec((1, H), lambda i: (0, 0)),
        ],
        out_specs=[
            pl.BlockSpec((BN, H), lambda i: (i, 0)),
            pl.BlockSpec((BN, H), lambda i: (i, 0)),
        ],
        out_shape=[
            jax.ShapeDtypeStruct((N, H), jnp.float32),
            jax.ShapeDtypeStruct((N, H), jnp.float32),
        ],
    )(x, Wl, bl.reshape(1, H), Wr, br.reshape(1, H))


def _edge_xform_body(ea_ref, we1_ref, we2_ref, ee1_ref, ee2_ref):
    ea = ea_ref[...]
    ee1_ref[...] = lax.dot_general(ea, we1_ref[...], (((1,), (1,)), ((), ())))
    ee2_ref[...] = lax.dot_general(ea, we2_ref[...], (((1,), (1,)), ((), ())))


def _edge_xform(ea, We1, We2):
    BE = 4000
    return pl.pallas_call(
        _edge_xform_body,
        grid=(E // BE,),
        in_specs=[
            pl.BlockSpec((BE, DE), lambda i: (i, 0)),
            pl.BlockSpec((H, DE), lambda i: (0, 0)),
            pl.BlockSpec((H, DE), lambda i: (0, 0)),
        ],
        out_specs=[
            pl.BlockSpec((BE, H), lambda i: (i, 0)),
            pl.BlockSpec((BE, H), lambda i: (i, 0)),
        ],
        out_shape=[
            jax.ShapeDtypeStruct((E, H), jnp.float32),
            jax.ShapeDtypeStruct((E, H), jnp.float32),
        ],
    )(ea, We1, We2)


def _sc_edge_pass(xl, xr, ee, ea, src, dst, att, with_ea):
    W = W1 if with_ea else W2
    mesh = plsc.VectorSubcoreMesh(core_axis_name="c", subcore_axis_name="s")

    ZR = 40

    def body(xl_hbm, xr_hbm, ee_hbm, ea_hbm, src_hbm, dst_hbm, att_hbm,
             out_hbm, accum,
             src_v0, dst_v0, xl_v0, xr_v0, ee_v0, prod_v0,
             src_v1, dst_v1, xl_v1, xr_v1, ee_v1, prod_v1,
             att_v, zb, g0a, g0b, g0c, g0d, g1a, g1b, g1c, g1d):
        bufs = [
            (src_v0, dst_v0, xl_v0, xr_v0, ee_v0, prod_v0, (g0a, g0b, g0c, g0d)),
            (src_v1, dst_v1, xl_v1, xr_v1, ee_v1, prod_v1, (g1a, g1b, g1c, g1d)),
        ]
        c = lax.axis_index("c")
        s = lax.axis_index("s")
        wid = s * NC + c
        zv = jnp.zeros((16,), jnp.float32)

        def zrow(i, carry):
            for k in range(W // 16):
                zb[i, pl.ds(16 * k, 16)] = zv
            return carry

        lax.fori_loop(0, ZR, zrow, 0)
        for r in range(RPS // ZR):
            pltpu.sync_copy(zb, accum.at[pl.ds(s * RPS + r * ZR, ZR)])
        pltpu.sync_copy(att_hbm, att_v)
        plsc.subcore_barrier()

        a0 = att_v[pl.ds(0, 16)]
        a1 = att_v[pl.ds(16, 16)]
        a2 = att_v[pl.ds(32, 16)]
        a3 = att_v[pl.ds(48, 16)]
        lane = lax.iota(jnp.int32, 16)
        s8 = lane ^ 8
        s4 = lane ^ 4
        s2 = lane ^ 2
        s1 = lane ^ 1
        c0 = jnp.where(lane == 0, 1.0, 0.0).astype(jnp.float32)
        c1 = jnp.where(lane == 1, 1.0, 0.0).astype(jnp.float32)
        base = wid * EPW
        tail_col = 80 if with_ea else 64

        def issue(j, b):
            src_v, dst_v, xl_v, xr_v, ee_v, prod_v, sems = bufs[b]
            off = base + j * C
            pltpu.sync_copy(src_hbm.at[pl.ds(off, C)], src_v)
            pltpu.sync_copy(dst_hbm.at[pl.ds(off, C)], dst_v)
            pltpu.async_copy(xl_hbm.at[src_v], xl_v, sems[0])
            pltpu.async_copy(xr_hbm.at[dst_v], xr_v, sems[1])
            pltpu.async_copy(ee_hbm.at[pl.ds(off, C), :], ee_v, sems[2])
            if with_ea:
                pltpu.async_copy(ea_hbm.at[pl.ds(off, C), :],
                                 prod_v.at[:, pl.ds(64, 16)], sems[3])

        def compute(b):
            src_v, dst_v, xl_v, xr_v, ee_v, prod_v, sems = bufs[b]
            pltpu.make_async_copy(xl_hbm.at[src_v], xl_v, sems[0]).wait()
            pltpu.make_async_copy(xr_hbm.at[dst_v], xr_v, sems[1]).wait()
            pltpu.make_async_copy(ee_hbm.at[pl.ds(0, C), :], ee_v,
                                  sems[2]).wait()
            if with_ea:
                pltpu.make_async_copy(ea_hbm.at[pl.ds(0, C), :],
                                      prod_v.at[:, pl.ds(64, 16)],
                                      sems[3]).wait()

            @plsc.parallel_loop(0, C, unroll=4)
            def edge_body(e):
                x0 = xl_v[e, pl.ds(0, 16)]
                x1 = xl_v[e, pl.ds(16, 16)]
                x2 = xl_v[e, pl.ds(32, 16)]
                x3 = xl_v[e, pl.ds(48, 16)]
                q0 = x0 + xr_v[e, pl.ds(0, 16)] + ee_v[e, pl.ds(0, 16)]
                q1 = x1 + xr_v[e, pl.ds(16, 16)] + ee_v[e, pl.ds(16, 16)]
                q2 = x2 + xr_v[e, pl.ds(32, 16)] + ee_v[e, pl.ds(32, 16)]
                q3 = x3 + xr_v[e, pl.ds(48, 16)] + ee_v[e, pl.ds(48, 16)]
                t = (_bfr(jnp.maximum(q0, 0.2 * q0)) * a0
                     + _bfr(jnp.maximum(q1, 0.2 * q1)) * a1
                     + _bfr(jnp.maximum(q2, 0.2 * q2)) * a2
                     + _bfr(jnp.maximum(q3, 0.2 * q3)) * a3)
                t = t + _shuf(t, s8)
                t = t + _shuf(t, s4)
                t = t + _shuf(t, s2)
                t = t + _shuf(t, s1)
                p = jnp.exp(t)
                prod_v[e, pl.ds(0, 16)] = p * x0
                prod_v[e, pl.ds(16, 16)] = p * x1
                prod_v[e, pl.ds(32, 16)] = p * x2
                prod_v[e, pl.ds(48, 16)] = p * x3
                if with_ea:
                    prod_v[e, pl.ds(tail_col, 16)] = c0 + c1 * p
                else:
                    prod_v[e, pl.ds(tail_col, 16)] = c0 * p

            pltpu.sync_copy(prod_v, accum.at[dst_v], add=True)

        issue(0, 0)

        def pair_body(jp, carry):
            j0 = 2 * jp
            issue(j0 + 1, 1)
            compute(0)
            issue(j0 + 2, 0)
            compute(1)
            return carry

        lax.fori_loop(0, (NCHUNK - 1) // 2, pair_body, 0)
        compute(0)
        plsc.subcore_barrier()
        pltpu.sync_copy(accum.at[pl.ds(s * RPS, RPS)],
                        out_hbm.at[c, pl.ds(s * RPS, RPS)])

    dbuf = [
        pltpu.VMEM((C,), jnp.int32),
        pltpu.VMEM((C,), jnp.int32),
        pltpu.VMEM((C, H), jnp.float32),
        pltpu.VMEM((C, H), jnp.float32),
        pltpu.VMEM((C, H), jnp.float32),
        pltpu.VMEM((C, W), jnp.float32),
    ]
    scratch = (
        [pltpu.VMEM_SHARED((NPAD, W), jnp.float32)]
        + dbuf + dbuf
        + [pltpu.VMEM((H,), jnp.float32), pltpu.VMEM((ZR, W), jnp.float32)]
        + [pltpu.SemaphoreType.DMA] * 8
    )
    f = pl.kernel(
        body,
        out_type=jax.ShapeDtypeStruct((NC, NPAD, W), jnp.float32),
        mesh=mesh,
        scratch_types=scratch,
        compiler_params=pltpu.CompilerParams(use_tc_tiling_on_sc=False),
    )
    return f(xl, xr, ee, ea, src, dst, att)


def _fin1_body(part_ref, xl_ref, xr_ref, att_ref, bias_ref, we1_ref,
               wl2_ref, bl2_ref, wr2_ref, br2_ref, we2_ref,
               xl2_ref, xr2_ref, lee2_ref):
    acc = part_ref[0] + part_ref[1]
    xl = xl_ref[...]
    xr = xr_ref[...]
    ea_sum = acc[:, 64:80]
    cnt = acc[:, 80:81]
    psum = acc[:, 81:82]
    la = ea_sum / jnp.maximum(cnt, 1.0)
    lee1 = lax.dot_general(la, we1_ref[...], (((1,), (1,)), ((), ())))
    q = xl + xr + lee1
    t = jnp.maximum(q, 0.2 * q)
    logit = jnp.sum(_bfr(t) * _bfr(att_ref[...]), axis=1, keepdims=True)
    p_loop = jnp.exp(logit)
    denom = psum + p_loop
    out1 = (acc[:, 0:64] + p_loop * xl) / denom + bias_ref[...]
    h1 = jnp.maximum(out1, 0.0)
    xl2_ref[...] = lax.dot_general(h1, wl2_ref[...], (((1,), (1,)), ((), ()))) + bl2_ref[...]
    xr2_ref[...] = lax.dot_general(h1, wr2_ref[...], (((1,), (1,)), ((), ()))) + br2_ref[...]
    lee2_ref[...] = lax.dot_general(la, we2_ref[...], (((1,), (1,)), ((), ())))


def _fin1(part, xl1, xr1, att1, bias1, We1, Wl2, bl2, Wr2, br2, We2):
    BN = 1000
    return pl.pallas_call(
        _fin1_body,
        grid=(N // BN,),
        in_specs=[
            pl.BlockSpec((NC, BN, W1), lambda i: (0, i, 0)),
            pl.BlockSpec((BN, H), lambda i: (i, 0)),
            pl.BlockSpec((BN, H), lambda i: (i, 0)),
            pl.BlockSpec((1, H), lambda i: (0, 0)),
            pl.BlockSpec((1, H), lambda i: (0, 0)),
            pl.BlockSpec((H, DE), lambda i: (0, 0)),
            pl.BlockSpec((H, H), lambda i: (0, 0)),
            pl.BlockSpec((1, H), lambda i: (0, 0)),
            pl.BlockSpec((H, H), lambda i: (0, 0)),
            pl.BlockSpec((1, H), lambda i: (0, 0)),
            pl.BlockSpec((H, DE), lambda i: (0, 0)),
        ],
        out_specs=[
            pl.BlockSpec((BN, H), lambda i: (i, 0)),
            pl.BlockSpec((BN, H), lambda i: (i, 0)),
            pl.BlockSpec((BN, H), lambda i: (i, 0)),
        ],
        out_shape=[
            jax.ShapeDtypeStruct((N, H), jnp.float32),
            jax.ShapeDtypeStruct((N, H), jnp.float32),
            jax.ShapeDtypeStruct((N, H), jnp.float32),
        ],
    )(part, xl1, xr1, att1.reshape(1, H), bias1.reshape(1, H), We1,
      Wl2, bl2.reshape(1, H), Wr2, br2.reshape(1, H), We2)


def _fin2_body(part_ref, xl_ref, xr_ref, lee_ref, att_ref, bias_ref,
               wn_ref, bn_ref, wg_ref, bg_ref, batch_ref,
               np_ref, gp_ref, gs_acc, gc_acc):
    i = pl.program_id(0)
    nblk = pl.num_programs(0)
    acc = part_ref[0] + part_ref[1]
    xl = xl_ref[...]
    psum = acc[:, 64:65]
    q = xl + xr_ref[...] + lee_ref[...]
    t = jnp.maximum(q, 0.2 * q)
    logit = jnp.sum(_bfr(t) * _bfr(att_ref[...]), axis=1, keepdims=True)
    p_loop = jnp.exp(logit)
    denom = psum + p_loop
    h2 = jnp.maximum((acc[:, 0:64] + p_loop * xl) / denom + bias_ref[...], 0.0)
    np_ref[...] = jnp.sum(_bfr(h2) * _bfr(wn_ref[...]), axis=1,
                          keepdims=True) + bn_ref[0, 0]

    seg = lax.broadcasted_iota(jnp.int32, (h2.shape[0], G), 1)
    onehot = (batch_ref[...] == seg).astype(jnp.float32)

    @pl.when(i == 0)
    def _():
        gs_acc[...] = jnp.zeros_like(gs_acc)
        gc_acc[...] = jnp.zeros_like(gc_acc)

    gs_acc[...] += lax.dot_general(onehot, h2, (((0,), (0,)), ((), ())), precision=lax.Precision.HIGHEST)
    gc_acc[...] += lax.dot_general(
        onehot, jnp.ones((h2.shape[0], 1), jnp.float32), (((0,), (0,)), ((), ())), precision=lax.Precision.HIGHEST)

    @pl.when(i == nblk - 1)
    def _():
        gmean = gs_acc[...] / jnp.maximum(gc_acc[...], 1.0)
        gp_ref[...] = jnp.sum(_bfr(gmean) * _bfr(wg_ref[...]), axis=1,
                              keepdims=True) + bg_ref[0, 0]


def _fin2(part, xl2, xr2, lee2, att2, bias2, Wn, bn, Wg, bg, batch):
    BN = 1000
    return pl.pallas_call(
        _fin2_body,
        grid=(N // BN,),
        in_specs=[
            pl.BlockSpec((NC, BN, W2), lambda i: (0, i, 0)),
            pl.BlockSpec((BN, H), lambda i: (i, 0)),
            pl.BlockSpec((BN, H), lambda i: (i, 0)),
            pl.BlockSpec((BN, H), lambda i: (i, 0)),
            pl.BlockSpec((1, H), lambda i: (0, 0)),
            pl.BlockSpec((1, H), lambda i: (0, 0)),
            pl.BlockSpec((1, H), lambda i: (0, 0)),
            pl.BlockSpec((1, 1), lambda i: (0, 0)),
            pl.BlockSpec((1, H), lambda i: (0, 0)),
            pl.BlockSpec((1, 1), lambda i: (0, 0)),
            pl.BlockSpec((BN, 1), lambda i: (i, 0)),
        ],
        out_specs=[
            pl.BlockSpec((BN, 1), lambda i: (i, 0)),
            pl.BlockSpec((G, 1), lambda i: (0, 0)),
        ],
        out_shape=[
            jax.ShapeDtypeStruct((N, 1), jnp.float32),
            jax.ShapeDtypeStruct((G, 1), jnp.float32),
        ],
        scratch_shapes=[
            pltpu.VMEM((G, H), jnp.float32),
            pltpu.VMEM((G, 1), jnp.float32),
        ],
    )(part, xl2, xr2, lee2, att2.reshape(1, H), bias2.reshape(1, H),
      Wn, bn.reshape(1, 1), Wg, bg.reshape(1, 1), batch.reshape(N, 1))


def kernel(x, edge_index, edge_attr, batch, Wl1, bl1, Wr1, br1, We1, att1,
           bias1, Wl2, bl2, Wr2, br2, We2, att2, bias2, Wn, bn, Wg, bg):
    src = edge_index[0].astype(jnp.int32)
    dst = edge_index[1].astype(jnp.int32)
    batch = batch.astype(jnp.int32)

    xl1, xr1 = _node_xform(x, Wl1, bl1, Wr1, br1)
    ee1, ee2 = _edge_xform(edge_attr, We1, We2)

    part1 = _sc_edge_pass(xl1, xr1, ee1, edge_attr, src, dst,
                          _bfr(att1), True)

    xl2, xr2, lee2 = _fin1(part1, xl1, xr1, att1, bias1, We1,
                           Wl2, bl2, Wr2, br2, We2)

    part2 = _sc_edge_pass(xl2, xr2, ee2, edge_attr, src, dst,
                          _bfr(att2), False)

    node_pred, graph_pred = _fin2(part2, xl2, xr2, lee2, att2, bias2,
                                  Wn, bn, Wg, bg, batch)
    return node_pred, graph_pred


# speedup vs baseline: 1.0635x; 1.0635x over previous
import jax
import jax.numpy as jnp
from jax import lax
from jax.experimental import pallas as pl
from jax.experimental.pallas import tpu as pltpu
from jax.experimental.pallas import tpu_sc as plsc

_GDN = lax.GatherDimensionNumbers(
    offset_dims=(), collapsed_slice_dims=(0,), start_index_map=(0,))


def _shuf(t, idx):
    return lax.gather(t, idx[:, None], _GDN, (1,),
                      mode=lax.GatherScatterMode.PROMISE_IN_BOUNDS)


def _bfr(v):
    i = lax.bitcast_convert_type(v, jnp.int32)
    r = (i + 0x7FFF + ((i >> 16) & 1)) & jnp.int32(-65536)
    return lax.bitcast_convert_type(r, jnp.float32)


def _bfr_fast(v):
    i = lax.bitcast_convert_type(v, jnp.int32)
    r = (i + 0x8000) & jnp.int32(-65536)
    return lax.bitcast_convert_type(r, jnp.float32)


N = 10000
E = 320000
DF = 128
H = 64
DE = 16
G = 64

NC = 2
NS = 16
NW = NC * NS
EPW = E // NW
C = 80
NCHUNK = EPW // C
NPAD = 10240
RPS = NPAD // NS

W1 = 96
W2 = 80


def _node_xform_body(x_ref, wl_ref, bl_ref, wr_ref, br_ref, xl_ref, xr_ref):
    x = x_ref[...]
    xl_ref[...] = lax.dot_general(x, wl_ref[...], (((1,), (1,)), ((), ()))) + bl_ref[...]
    xr_ref[...] = lax.dot_general(x, wr_ref[...], (((1,), (1,)), ((), ()))) + br_ref[...]


def _node_xform(x, Wl, bl, Wr, br):
    BN = 1000
    return pl.pallas_call(
        _node_xform_body,
        grid=(N // BN,),
        in_specs=[
            pl.BlockSpec((BN, DF), lambda i: (i, 0)),
            pl.BlockSpec((H, DF), lambda i: (0, 0)),
            pl.BlockSpec((1, H), lambda i: (0, 0)),
            pl.BlockSpec((H, DF), lambda i: (0, 0)),
            pl.BlockSpec((1, H), lambda i: (0, 0)),
        ],
        out_specs=[
            pl.BlockSpec((BN, H), lambda i: (i, 0)),
            pl.BlockSpec((BN, H), lambda i: (i, 0)),
        ],
        out_shape=[
            jax.ShapeDtypeStruct((N, H), jnp.float32),
            jax.ShapeDtypeStruct((N, H), jnp.float32),
        ],
    )(x, Wl, bl.reshape(1, H), Wr, br.reshape(1, H))


def _edge_xform_body(ea_ref, we_ref, ee_ref):
    ee_ref[...] = lax.dot_general(
        ea_ref[...], we_ref[...], (((1,), (1,)), ((), ())))


def _edge_xform(ea, We):
    BE = 4000
    return pl.pallas_call(
        _edge_xform_body,
        grid=(E // BE,),
        in_specs=[
            pl.BlockSpec((BE, DE), lambda i: (i, 0)),
            pl.BlockSpec((H, DE), lambda i: (0, 0)),
        ],
        out_specs=pl.BlockSpec((BE, H), lambda i: (i, 0)),
        out_shape=jax.ShapeDtypeStruct((E, H), jnp.float32),
    )(ea, We)


def _sc_edge_pass(xl, xr, ee, ea, src, dst, att, with_ea):
    W = W1 if with_ea else W2
    mesh = plsc.VectorSubcoreMesh(core_axis_name="c", subcore_axis_name="s")

    ZR = 40

    def body(xl_hbm, xr_hbm, ee_hbm, ea_hbm, src_hbm, dst_hbm, att_hbm,
             out_hbm, accum,
             src_v0, dst_v0, xl_v0, xr_v0, ee_v0, prod_v0,
             src_v1, dst_v1, xl_v1, xr_v1, ee_v1, prod_v1,
             att_v, zb, g0a, g0b, g0c, g0d, g1a, g1b, g1c, g1d):
        bufs = [
            (src_v0, dst_v0, xl_v0, xr_v0, ee_v0, prod_v0, (g0a, g0b, g0c, g0d)),
            (src_v1, dst_v1, xl_v1, xr_v1, ee_v1, prod_v1, (g1a, g1b, g1c, g1d)),
        ]
        c = lax.axis_index("c")
        s = lax.axis_index("s")
        wid = s * NC + c
        zv = jnp.zeros((16,), jnp.float32)

        def zrow(i, carry):
            for k in range(W // 16):
                zb[i, pl.ds(16 * k, 16)] = zv
            return carry

        lax.fori_loop(0, ZR, zrow, 0)
        for r in range(RPS // ZR):
            pltpu.sync_copy(zb, accum.at[pl.ds(s * RPS + r * ZR, ZR)])
        pltpu.sync_copy(att_hbm, att_v)
        plsc.subcore_barrier()

        a0 = att_v[pl.ds(0, 16)]
        a1 = att_v[pl.ds(16, 16)]
        a2 = att_v[pl.ds(32, 16)]
        a3 = att_v[pl.ds(48, 16)]
        lane = lax.iota(jnp.int32, 16)
        s8 = lane ^ 8
        s4 = lane ^ 4
        s2 = lane ^ 2
        s1 = lane ^ 1
        c0 = jnp.where(lane == 0, 1.0, 0.0).astype(jnp.float32)
        c1 = jnp.where(lane == 1, 1.0, 0.0).astype(jnp.float32)
        base = wid * EPW
        tail_col = 80 if with_ea else 64

        def issue(j, b):
            src_v, dst_v, xl_v, xr_v, ee_v, prod_v, sems = bufs[b]
            off = base + j * C
            pltpu.sync_copy(src_hbm.at[pl.ds(off, C)], src_v)
            pltpu.sync_copy(dst_hbm.at[pl.ds(off, C)], dst_v)
            pltpu.async_copy(xl_hbm.at[src_v], xl_v, sems[0])
            pltpu.async_copy(xr_hbm.at[dst_v], xr_v, sems[1])
            pltpu.async_copy(ee_hbm.at[pl.ds(off, C), :], ee_v, sems[2])
            if with_ea:
                pltpu.async_copy(ea_hbm.at[pl.ds(off, C), :],
                                 prod_v.at[:, pl.ds(64, 16)], sems[3])

        def compute(b):
            src_v, dst_v, xl_v, xr_v, ee_v, prod_v, sems = bufs[b]
            pltpu.make_async_copy(xl_hbm.at[src_v], xl_v, sems[0]).wait()
            pltpu.make_async_copy(xr_hbm.at[dst_v], xr_v, sems[1]).wait()
            pltpu.make_async_copy(ee_hbm.at[pl.ds(0, C), :], ee_v,
                                  sems[2]).wait()
            if with_ea:
                pltpu.make_async_copy(ea_hbm.at[pl.ds(0, C), :],
                                      prod_v.at[:, pl.ds(64, 16)],
                                      sems[3]).wait()

            @plsc.parallel_loop(0, C, unroll=4)
            def edge_body(e):
                x0 = xl_v[e, pl.ds(0, 16)]
                x1 = xl_v[e, pl.ds(16, 16)]
                x2 = xl_v[e, pl.ds(32, 16)]
                x3 = xl_v[e, pl.ds(48, 16)]
                q0 = x0 + xr_v[e, pl.ds(0, 16)] + ee_v[e, pl.ds(0, 16)]
                q1 = x1 + xr_v[e, pl.ds(16, 16)] + ee_v[e, pl.ds(16, 16)]
                q2 = x2 + xr_v[e, pl.ds(32, 16)] + ee_v[e, pl.ds(32, 16)]
                q3 = x3 + xr_v[e, pl.ds(48, 16)] + ee_v[e, pl.ds(48, 16)]
                t = (_bfr_fast(jnp.maximum(q0, 0.2 * q0)) * a0
                     + _bfr_fast(jnp.maximum(q1, 0.2 * q1)) * a1
                     + _bfr_fast(jnp.maximum(q2, 0.2 * q2)) * a2
                     + _bfr_fast(jnp.maximum(q3, 0.2 * q3)) * a3)
                t = t + _shuf(t, s8)
                t = t + _shuf(t, s4)
                t = t + _shuf(t, s2)
                t = t + _shuf(t, s1)
                p = jnp.exp(t)
                prod_v[e, pl.ds(0, 16)] = p * x0
                prod_v[e, pl.ds(16, 16)] = p * x1
                prod_v[e, pl.ds(32, 16)] = p * x2
                prod_v[e, pl.ds(48, 16)] = p * x3
                if with_ea:
                    prod_v[e, pl.ds(tail_col, 16)] = c0 + c1 * p
                else:
                    prod_v[e, pl.ds(tail_col, 16)] = c0 * p

            pltpu.sync_copy(prod_v, accum.at[dst_v], add=True)

        issue(0, 0)

        def pair_body(jp, carry):
            j0 = 2 * jp
            issue(j0 + 1, 1)
            compute(0)
            issue(j0 + 2, 0)
            compute(1)
            return carry

        lax.fori_loop(0, (NCHUNK - 1) // 2, pair_body, 0)
        compute(0)
        plsc.subcore_barrier()
        pltpu.sync_copy(accum.at[pl.ds(s * RPS, RPS)],
                        out_hbm.at[c, pl.ds(s * RPS, RPS)])

    dbuf = [
        pltpu.VMEM((C,), jnp.int32),
        pltpu.VMEM((C,), jnp.int32),
        pltpu.VMEM((C, H), jnp.float32),
        pltpu.VMEM((C, H), jnp.float32),
        pltpu.VMEM((C, H), jnp.float32),
        pltpu.VMEM((C, W), jnp.float32),
    ]
    scratch = (
        [pltpu.VMEM_SHARED((NPAD, W), jnp.float32)]
        + dbuf + dbuf
        + [pltpu.VMEM((H,), jnp.float32), pltpu.VMEM((ZR, W), jnp.float32)]
        + [pltpu.SemaphoreType.DMA] * 8
    )
    f = pl.kernel(
        body,
        out_type=jax.ShapeDtypeStruct((NC, NPAD, W), jnp.float32),
        mesh=mesh,
        scratch_types=scratch,
        compiler_params=pltpu.CompilerParams(use_tc_tiling_on_sc=False),
    )
    return f(xl, xr, ee, ea, src, dst, att)


def _fin1_body(part_ref, xl_ref, xr_ref, att_ref, bias_ref, we1_ref,
               wl2_ref, bl2_ref, wr2_ref, br2_ref, we2_ref,
               xl2_ref, xr2_ref, lee2_ref):
    acc = part_ref[0] + part_ref[1]
    xl = xl_ref[...]
    xr = xr_ref[...]
    ea_sum = acc[:, 64:80]
    cnt = acc[:, 80:81]
    psum = acc[:, 81:82]
    la = ea_sum / jnp.maximum(cnt, 1.0)
    lee1 = lax.dot_general(la, we1_ref[...], (((1,), (1,)), ((), ())))
    q = xl + xr + lee1
    t = jnp.maximum(q, 0.2 * q)
    logit = jnp.sum(_bfr(t) * _bfr(att_ref[...]), axis=1, keepdims=True)
    p_loop = jnp.exp(logit)
    denom = psum + p_loop
    out1 = (acc[:, 0:64] + p_loop * xl) / denom + bias_ref[...]
    h1 = jnp.maximum(out1, 0.0)
    xl2_ref[...] = lax.dot_general(h1, wl2_ref[...], (((1,), (1,)), ((), ()))) + bl2_ref[...]
    xr2_ref[...] = lax.dot_general(h1, wr2_ref[...], (((1,), (1,)), ((), ()))) + br2_ref[...]
    lee2_ref[...] = lax.dot_general(la, we2_ref[...], (((1,), (1,)), ((), ())))


def _fin1(part, xl1, xr1, att1, bias1, We1, Wl2, bl2, Wr2, br2, We2):
    BN = 1000
    return pl.pallas_call(
        _fin1_body,
        grid=(N // BN,),
        in_specs=[
            pl.BlockSpec((NC, BN, W1), lambda i: (0, i, 0)),
            pl.BlockSpec((BN, H), lambda i: (i, 0)),
            pl.BlockSpec((BN, H), lambda i: (i, 0)),
            pl.BlockSpec((1, H), lambda i: (0, 0)),
            pl.BlockSpec((1, H), lambda i: (0, 0)),
            pl.BlockSpec((H, DE), lambda i: (0, 0)),
            pl.BlockSpec((H, H), lambda i: (0, 0)),
            pl.BlockSpec((1, H), lambda i: (0, 0)),
            pl.BlockSpec((H, H), lambda i: (0, 0)),
            pl.BlockSpec((1, H), lambda i: (0, 0)),
            pl.BlockSpec((H, DE), lambda i: (0, 0)),
        ],
        out_specs=[
            pl.BlockSpec((BN, H), lambda i: (i, 0)),
            pl.BlockSpec((BN, H), lambda i: (i, 0)),
            pl.BlockSpec((BN, H), lambda i: (i, 0)),
        ],
        out_shape=[
            jax.ShapeDtypeStruct((N, H), jnp.float32),
            jax.ShapeDtypeStruct((N, H), jnp.float32),
            jax.ShapeDtypeStruct((N, H), jnp.float32),
        ],
    )(part, xl1, xr1, att1.reshape(1, H), bias1.reshape(1, H), We1,
      Wl2, bl2.reshape(1, H), Wr2, br2.reshape(1, H), We2)


def _fin2_body(part_ref, xl_ref, xr_ref, lee_ref, att_ref, bias_ref,
               wn_ref, bn_ref, wg_ref, bg_ref, batch_ref,
               np_ref, gp_ref, gs_acc, gc_acc):
    i = pl.program_id(0)
    nblk = pl.num_programs(0)
    acc = part_ref[0] + part_ref[1]
    xl = xl_ref[...]
    psum = acc[:, 64:65]
    q = xl + xr_ref[...] + lee_ref[...]
    t = jnp.maximum(q, 0.2 * q)
    logit = jnp.sum(_bfr(t) * _bfr(att_ref[...]), axis=1, keepdims=True)
    p_loop = jnp.exp(logit)
    denom = psum + p_loop
    h2 = jnp.maximum((acc[:, 0:64] + p_loop * xl) / denom + bias_ref[...], 0.0)
    np_ref[...] = jnp.sum(_bfr(h2) * _bfr(wn_ref[...]), axis=1,
                          keepdims=True) + bn_ref[0, 0]

    seg = lax.broadcasted_iota(jnp.int32, (h2.shape[0], G), 1)
    onehot = (batch_ref[...] == seg).astype(jnp.float32)

    @pl.when(i == 0)
    def _():
        gs_acc[...] = jnp.zeros_like(gs_acc)
        gc_acc[...] = jnp.zeros_like(gc_acc)

    gs_acc[...] += lax.dot_general(onehot, h2, (((0,), (0,)), ((), ())), precision=lax.Precision.HIGHEST)
    gc_acc[...] += lax.dot_general(
        onehot, jnp.ones((h2.shape[0], 1), jnp.float32), (((0,), (0,)), ((), ())), precision=lax.Precision.HIGHEST)

    @pl.when(i == nblk - 1)
    def _():
        gmean = gs_acc[...] / jnp.maximum(gc_acc[...], 1.0)
        gp_ref[...] = jnp.sum(_bfr(gmean) * _bfr(wg_ref[...]), axis=1,
                              keepdims=True) + bg_ref[0, 0]


def _fin2(part, xl2, xr2, lee2, att2, bias2, Wn, bn, Wg, bg, batch):
    BN = 1000
    return pl.pallas_call(
        _fin2_body,
        grid=(N // BN,),
        in_specs=[
            pl.BlockSpec((NC, BN, W2), lambda i: (0, i, 0)),
            pl.BlockSpec((BN, H), lambda i: (i, 0)),
            pl.BlockSpec((BN, H), lambda i: (i, 0)),
            pl.BlockSpec((BN, H), lambda i: (i, 0)),
            pl.BlockSpec((1, H), lambda i: (0, 0)),
            pl.BlockSpec((1, H), lambda i: (0, 0)),
            pl.BlockSpec((1, H), lambda i: (0, 0)),
            pl.BlockSpec((1, 1), lambda i: (0, 0)),
            pl.BlockSpec((1, H), lambda i: (0, 0)),
            pl.BlockSpec((1, 1), lambda i: (0, 0)),
            pl.BlockSpec((BN, 1), lambda i: (i, 0)),
        ],
        out_specs=[
            pl.BlockSpec((BN, 1), lambda i: (i, 0)),
            pl.BlockSpec((G, 1), lambda i: (0, 0)),
        ],
        out_shape=[
            jax.ShapeDtypeStruct((N, 1), jnp.float32),
            jax.ShapeDtypeStruct((G, 1), jnp.float32),
        ],
        scratch_shapes=[
            pltpu.VMEM((G, H), jnp.float32),
            pltpu.VMEM((G, 1), jnp.float32),
        ],
    )(part, xl2, xr2, lee2, att2.reshape(1, H), bias2.reshape(1, H),
      Wn, bn.reshape(1, 1), Wg, bg.reshape(1, 1), batch.reshape(N, 1))


def kernel(x, edge_index, edge_attr, batch, Wl1, bl1, Wr1, br1, We1, att1,
           bias1, Wl2, bl2, Wr2, br2, We2, att2, bias2, Wn, bn, Wg, bg):
    src = edge_index[0].astype(jnp.int32)
    dst = edge_index[1].astype(jnp.int32)
    batch = batch.astype(jnp.int32)

    xl1, xr1 = _node_xform(x, Wl1, bl1, Wr1, br1)
    ee1 = _edge_xform(edge_attr, We1)

    part1 = _sc_edge_pass(xl1, xr1, ee1, edge_attr, src, dst,
                          _bfr(att1), True)

    ee2 = _edge_xform(edge_attr, We2)

    xl2, xr2, lee2 = _fin1(part1, xl1, xr1, att1, bias1, We1,
                           Wl2, bl2, Wr2, br2, We2)

    part2 = _sc_edge_pass(xl2, xr2, ee2, edge_attr, src, dst,
                          _bfr(att2), False)

    node_pred, graph_pred = _fin2(part2, xl2, xr2, lee2, att2, bias2,
                                  Wn, bn, Wg, bg, batch)
    return node_pred, graph_pred


# speedup vs baseline: 1.1607x; 1.0914x over previous
import jax
import jax.numpy as jnp
from jax import lax
from jax.experimental import pallas as pl
from jax.experimental.pallas import tpu as pltpu
from jax.experimental.pallas import tpu_sc as plsc

_GDN = lax.GatherDimensionNumbers(
    offset_dims=(), collapsed_slice_dims=(0,), start_index_map=(0,))


def _shuf(t, idx):
    return lax.gather(t, idx[:, None], _GDN, (1,),
                      mode=lax.GatherScatterMode.PROMISE_IN_BOUNDS)


def _bfr(v):
    i = lax.bitcast_convert_type(v, jnp.int32)
    r = (i + 0x7FFF + ((i >> 16) & 1)) & jnp.int32(-65536)
    return lax.bitcast_convert_type(r, jnp.float32)


def _bfr_fast(v):
    i = lax.bitcast_convert_type(v, jnp.int32)
    r = (i + 0x8000) & jnp.int32(-65536)
    return lax.bitcast_convert_type(r, jnp.float32)


N = 10000
E = 320000
DF = 128
H = 64
DE = 16
G = 64

NC = 2
NS = 16
NW = NC * NS
EPW = E // NW
C = 80
NCHUNK = EPW // C
NPAD = 10240
RPS = NPAD // NS

W1 = 96
W2 = 80


def _node_xform_body(x_ref, wl_ref, bl_ref, wr_ref, br_ref, xl_ref, xr_ref):
    x = x_ref[...]
    xl_ref[...] = lax.dot_general(x, wl_ref[...], (((1,), (1,)), ((), ()))) + bl_ref[...]
    xr_ref[...] = lax.dot_general(x, wr_ref[...], (((1,), (1,)), ((), ()))) + br_ref[...]


def _node_xform(x, Wl, bl, Wr, br):
    BN = 1000
    return pl.pallas_call(
        _node_xform_body,
        grid=(N // BN,),
        in_specs=[
            pl.BlockSpec((BN, DF), lambda i: (i, 0)),
            pl.BlockSpec((H, DF), lambda i: (0, 0)),
            pl.BlockSpec((1, H), lambda i: (0, 0)),
            pl.BlockSpec((H, DF), lambda i: (0, 0)),
            pl.BlockSpec((1, H), lambda i: (0, 0)),
        ],
        out_specs=[
            pl.BlockSpec((BN, H), lambda i: (i, 0)),
            pl.BlockSpec((BN, H), lambda i: (i, 0)),
        ],
        out_shape=[
            jax.ShapeDtypeStruct((N, H), jnp.float32),
            jax.ShapeDtypeStruct((N, H), jnp.float32),
        ],
    )(x, Wl, bl.reshape(1, H), Wr, br.reshape(1, H))


def _edge_xform_body(ea_ref, we_ref, ee_ref):
    ee_ref[...] = lax.dot_general(
        ea_ref[...], we_ref[...], (((1,), (1,)), ((), ())))


def _edge_xform(ea, We):
    BE = 4000
    return pl.pallas_call(
        _edge_xform_body,
        grid=(E // BE,),
        in_specs=[
            pl.BlockSpec((BE, DE), lambda i: (i, 0)),
            pl.BlockSpec((H, DE), lambda i: (0, 0)),
        ],
        out_specs=pl.BlockSpec((BE, H), lambda i: (i, 0)),
        out_shape=jax.ShapeDtypeStruct((E, H), jnp.float32),
    )(ea, We)


def _sc_edge_pass(xl, xr, ee, ea, src, dst, att, with_ea):
    W = W1 if with_ea else W2
    mesh = plsc.VectorSubcoreMesh(core_axis_name="c", subcore_axis_name="s")

    ZR = 40

    def body(xl_hbm, xr_hbm, ee_hbm, ea_hbm, src_hbm, dst_hbm, att_hbm,
             out_hbm, accum,
             src_v0, dst_v0, xl_v0, xr_v0, ee_v0, ea_v0, prod_v0, sdst_v0,
             src_v1, dst_v1, xl_v1, xr_v1, ee_v1, ea_v1, prod_v1, sdst_v1,
             att_v, zb, g0a, g0b, g0c, g0d, g1a, g1b, g1c, g1d, sc0, sc1):
        bufs = [
            (src_v0, dst_v0, xl_v0, xr_v0, ee_v0, ea_v0, prod_v0, sdst_v0,
             (g0a, g0b, g0c, g0d), sc0),
            (src_v1, dst_v1, xl_v1, xr_v1, ee_v1, ea_v1, prod_v1, sdst_v1,
             (g1a, g1b, g1c, g1d), sc1),
        ]
        c = lax.axis_index("c")
        s = lax.axis_index("s")
        wid = s * NC + c
        zv = jnp.zeros((16,), jnp.float32)

        def zrow(i, carry):
            for k in range(W // 16):
                zb[i, pl.ds(16 * k, 16)] = zv
            return carry

        lax.fori_loop(0, ZR, zrow, 0)
        for r in range(RPS // ZR):
            pltpu.sync_copy(zb, accum.at[pl.ds(s * RPS + r * ZR, ZR)])
        pltpu.sync_copy(att_hbm, att_v)
        plsc.subcore_barrier()

        a0 = att_v[pl.ds(0, 16)]
        a1 = att_v[pl.ds(16, 16)]
        a2 = att_v[pl.ds(32, 16)]
        a3 = att_v[pl.ds(48, 16)]
        lane = lax.iota(jnp.int32, 16)
        s8 = lane ^ 8
        s4 = lane ^ 4
        s2 = lane ^ 2
        s1 = lane ^ 1
        c0 = jnp.where(lane == 0, 1.0, 0.0).astype(jnp.float32)
        c1 = jnp.where(lane == 1, 1.0, 0.0).astype(jnp.float32)
        base = wid * EPW
        tail_col = 80 if with_ea else 64

        def issue(j, b):
            src_v, dst_v, xl_v, xr_v, ee_v, ea_v, prod_v, sdst_v, sems, scs = bufs[b]
            off = base + j * C
            pltpu.sync_copy(src_hbm.at[pl.ds(off, C)], src_v)
            pltpu.sync_copy(dst_hbm.at[pl.ds(off, C)], dst_v)
            pltpu.async_copy(xl_hbm.at[src_v], xl_v, sems[0])
            pltpu.async_copy(xr_hbm.at[dst_v], xr_v, sems[1])
            pltpu.async_copy(ee_hbm.at[pl.ds(off, C), :], ee_v, sems[2])
            if with_ea:
                pltpu.async_copy(ea_hbm.at[pl.ds(off, C), :], ea_v, sems[3])

        def wait_scat(b):
            _, _, _, _, _, _, prod_v, sdst_v, _, scs = bufs[b]
            pltpu.make_async_copy(prod_v, accum.at[sdst_v], scs).wait()

        def compute(b, have_scat):
            src_v, dst_v, xl_v, xr_v, ee_v, ea_v, prod_v, sdst_v, sems, scs = bufs[b]
            pltpu.make_async_copy(xl_hbm.at[src_v], xl_v, sems[0]).wait()
            pltpu.make_async_copy(xr_hbm.at[dst_v], xr_v, sems[1]).wait()
            pltpu.make_async_copy(ee_hbm.at[pl.ds(0, C), :], ee_v,
                                  sems[2]).wait()
            if with_ea:
                pltpu.make_async_copy(ea_hbm.at[pl.ds(0, C), :], ea_v,
                                      sems[3]).wait()

            @pl.when(have_scat)
            def _():
                wait_scat(b)

            @plsc.parallel_loop(0, C, unroll=4)
            def edge_body(e):
                x0 = xl_v[e, pl.ds(0, 16)]
                x1 = xl_v[e, pl.ds(16, 16)]
                x2 = xl_v[e, pl.ds(32, 16)]
                x3 = xl_v[e, pl.ds(48, 16)]
                q0 = x0 + xr_v[e, pl.ds(0, 16)] + ee_v[e, pl.ds(0, 16)]
                q1 = x1 + xr_v[e, pl.ds(16, 16)] + ee_v[e, pl.ds(16, 16)]
                q2 = x2 + xr_v[e, pl.ds(32, 16)] + ee_v[e, pl.ds(32, 16)]
                q3 = x3 + xr_v[e, pl.ds(48, 16)] + ee_v[e, pl.ds(48, 16)]
                t = (_bfr_fast(jnp.maximum(q0, 0.2 * q0)) * a0
                     + _bfr_fast(jnp.maximum(q1, 0.2 * q1)) * a1
                     + _bfr_fast(jnp.maximum(q2, 0.2 * q2)) * a2
                     + _bfr_fast(jnp.maximum(q3, 0.2 * q3)) * a3)
                t = t + _shuf(t, s8)
                t = t + _shuf(t, s4)
                t = t + _shuf(t, s2)
                t = t + _shuf(t, s1)
                p = jnp.exp(t)
                prod_v[e, pl.ds(0, 16)] = p * x0
                prod_v[e, pl.ds(16, 16)] = p * x1
                prod_v[e, pl.ds(32, 16)] = p * x2
                prod_v[e, pl.ds(48, 16)] = p * x3
                if with_ea:
                    prod_v[e, pl.ds(64, 16)] = ea_v[e, :]
                    prod_v[e, pl.ds(tail_col, 16)] = c0 + c1 * p
                else:
                    prod_v[e, pl.ds(tail_col, 16)] = c0 * p

            for k in range(C // 16):
                sdst_v[pl.ds(16 * k, 16)] = dst_v[pl.ds(16 * k, 16)]
            pltpu.async_copy(prod_v, accum.at[sdst_v], scs, add=True)

        issue(0, 0)

        def pair_body(jp, carry):
            j0 = 2 * jp
            issue(j0 + 1, 1)
            compute(0, jp > 0)
            issue(j0 + 2, 0)
            compute(1, jp > 0)
            return carry

        lax.fori_loop(0, (NCHUNK - 1) // 2, pair_body, 0)
        compute(0, jnp.bool_(True))
        wait_scat(0)
        wait_scat(1)
        plsc.subcore_barrier()
        pltpu.sync_copy(accum.at[pl.ds(s * RPS, RPS)],
                        out_hbm.at[c, pl.ds(s * RPS, RPS)])

    dbuf = [
        pltpu.VMEM((C,), jnp.int32),
        pltpu.VMEM((C,), jnp.int32),
        pltpu.VMEM((C, H), jnp.float32),
        pltpu.VMEM((C, H), jnp.float32),
        pltpu.VMEM((C, H), jnp.float32),
        pltpu.VMEM((C, DE), jnp.float32),
        pltpu.VMEM((C, W), jnp.float32),
        pltpu.VMEM((C,), jnp.int32),
    ]
    scratch = (
        [pltpu.VMEM_SHARED((NPAD, W), jnp.float32)]
        + dbuf + dbuf
        + [pltpu.VMEM((H,), jnp.float32), pltpu.VMEM((ZR, W), jnp.float32)]
        + [pltpu.SemaphoreType.DMA] * 10
    )
    f = pl.kernel(
        body,
        out_type=jax.ShapeDtypeStruct((NC, NPAD, W), jnp.float32),
        mesh=mesh,
        scratch_types=scratch,
        compiler_params=pltpu.CompilerParams(use_tc_tiling_on_sc=False),
    )
    return f(xl, xr, ee, ea, src, dst, att)


def _fin1_body(part_ref, xl_ref, xr_ref, att_ref, bias_ref, we1_ref,
               wl2_ref, bl2_ref, wr2_ref, br2_ref, we2_ref,
               xl2_ref, xr2_ref, lee2_ref):
    acc = part_ref[0] + part_ref[1]
    xl = xl_ref[...]
    xr = xr_ref[...]
    ea_sum = acc[:, 64:80]
    cnt = acc[:, 80:81]
    psum = acc[:, 81:82]
    la = ea_sum / jnp.maximum(cnt, 1.0)
    lee1 = lax.dot_general(la, we1_ref[...], (((1,), (1,)), ((), ())))
    q = xl + xr + lee1
    t = jnp.maximum(q, 0.2 * q)
    logit = jnp.sum(_bfr(t) * _bfr(att_ref[...]), axis=1, keepdims=True)
    p_loop = jnp.exp(logit)
    denom = psum + p_loop
    out1 = (acc[:, 0:64] + p_loop * xl) / denom + bias_ref[...]
    h1 = jnp.maximum(out1, 0.0)
    xl2_ref[...] = lax.dot_general(h1, wl2_ref[...], (((1,), (1,)), ((), ()))) + bl2_ref[...]
    xr2_ref[...] = lax.dot_general(h1, wr2_ref[...], (((1,), (1,)), ((), ()))) + br2_ref[...]
    lee2_ref[...] = lax.dot_general(la, we2_ref[...], (((1,), (1,)), ((), ())))


def _fin1(part, xl1, xr1, att1, bias1, We1, Wl2, bl2, Wr2, br2, We2):
    BN = 1000
    return pl.pallas_call(
        _fin1_body,
        grid=(N // BN,),
        in_specs=[
            pl.BlockSpec((NC, BN, W1), lambda i: (0, i, 0)),
            pl.BlockSpec((BN, H), lambda i: (i, 0)),
            pl.BlockSpec((BN, H), lambda i: (i, 0)),
            pl.BlockSpec((1, H), lambda i: (0, 0)),
            pl.BlockSpec((1, H), lambda i: (0, 0)),
            pl.BlockSpec((H, DE), lambda i: (0, 0)),
            pl.BlockSpec((H, H), lambda i: (0, 0)),
            pl.BlockSpec((1, H), lambda i: (0, 0)),
            pl.BlockSpec((H, H), lambda i: (0, 0)),
            pl.BlockSpec((1, H), lambda i: (0, 0)),
            pl.BlockSpec((H, DE), lambda i: (0, 0)),
        ],
        out_specs=[
            pl.BlockSpec((BN, H), lambda i: (i, 0)),
            pl.BlockSpec((BN, H), lambda i: (i, 0)),
            pl.BlockSpec((BN, H), lambda i: (i, 0)),
        ],
        out_shape=[
            jax.ShapeDtypeStruct((N, H), jnp.float32),
            jax.ShapeDtypeStruct((N, H), jnp.float32),
            jax.ShapeDtypeStruct((N, H), jnp.float32),
        ],
    )(part, xl1, xr1, att1.reshape(1, H), bias1.reshape(1, H), We1,
      Wl2, bl2.reshape(1, H), Wr2, br2.reshape(1, H), We2)


def _fin2_body(part_ref, xl_ref, xr_ref, lee_ref, att_ref, bias_ref,
               wn_ref, bn_ref, wg_ref, bg_ref, batch_ref,
               np_ref, gp_ref, gs_acc, gc_acc):
    i = pl.program_id(0)
    nblk = pl.num_programs(0)
    acc = part_ref[0] + part_ref[1]
    xl = xl_ref[...]
    psum = acc[:, 64:65]
    q = xl + xr_ref[...] + lee_ref[...]
    t = jnp.maximum(q, 0.2 * q)
    logit = jnp.sum(_bfr(t) * _bfr(att_ref[...]), axis=1, keepdims=True)
    p_loop = jnp.exp(logit)
    denom = psum + p_loop
    h2 = jnp.maximum((acc[:, 0:64] + p_loop * xl) / denom + bias_ref[...], 0.0)
    np_ref[...] = jnp.sum(_bfr(h2) * _bfr(wn_ref[...]), axis=1,
                          keepdims=True) + bn_ref[0, 0]

    seg = lax.broadcasted_iota(jnp.int32, (h2.shape[0], G), 1)
    onehot = (batch_ref[...] == seg).astype(jnp.float32)

    @pl.when(i == 0)
    def _():
        gs_acc[...] = jnp.zeros_like(gs_acc)
        gc_acc[...] = jnp.zeros_like(gc_acc)

    gs_acc[...] += lax.dot_general(onehot, h2, (((0,), (0,)), ((), ())), precision=lax.Precision.HIGHEST)
    gc_acc[...] += lax.dot_general(
        onehot, jnp.ones((h2.shape[0], 1), jnp.float32), (((0,), (0,)), ((), ())), precision=lax.Precision.HIGHEST)

    @pl.when(i == nblk - 1)
    def _():
        gmean = gs_acc[...] / jnp.maximum(gc_acc[...], 1.0)
        gp_ref[...] = jnp.sum(_bfr(gmean) * _bfr(wg_ref[...]), axis=1,
                              keepdims=True) + bg_ref[0, 0]


def _fin2(part, xl2, xr2, lee2, att2, bias2, Wn, bn, Wg, bg, batch):
    BN = 1000
    return pl.pallas_call(
        _fin2_body,
        grid=(N // BN,),
        in_specs=[
            pl.BlockSpec((NC, BN, W2), lambda i: (0, i, 0)),
            pl.BlockSpec((BN, H), lambda i: (i, 0)),
            pl.BlockSpec((BN, H), lambda i: (i, 0)),
            pl.BlockSpec((BN, H), lambda i: (i, 0)),
            pl.BlockSpec((1, H), lambda i: (0, 0)),
            pl.BlockSpec((1, H), lambda i: (0, 0)),
            pl.BlockSpec((1, H), lambda i: (0, 0)),
            pl.BlockSpec((1, 1), lambda i: (0, 0)),
            pl.BlockSpec((1, H), lambda i: (0, 0)),
            pl.BlockSpec((1, 1), lambda i: (0, 0)),
            pl.BlockSpec((BN, 1), lambda i: (i, 0)),
        ],
        out_specs=[
            pl.BlockSpec((BN, 1), lambda i: (i, 0)),
            pl.BlockSpec((G, 1), lambda i: (0, 0)),
        ],
        out_shape=[
            jax.ShapeDtypeStruct((N, 1), jnp.float32),
            jax.ShapeDtypeStruct((G, 1), jnp.float32),
        ],
        scratch_shapes=[
            pltpu.VMEM((G, H), jnp.float32),
            pltpu.VMEM((G, 1), jnp.float32),
        ],
    )(part, xl2, xr2, lee2, att2.reshape(1, H), bias2.reshape(1, H),
      Wn, bn.reshape(1, 1), Wg, bg.reshape(1, 1), batch.reshape(N, 1))


def kernel(x, edge_index, edge_attr, batch, Wl1, bl1, Wr1, br1, We1, att1,
           bias1, Wl2, bl2, Wr2, br2, We2, att2, bias2, Wn, bn, Wg, bg):
    src = edge_index[0].astype(jnp.int32)
    dst = edge_index[1].astype(jnp.int32)
    batch = batch.astype(jnp.int32)

    xl1, xr1 = _node_xform(x, Wl1, bl1, Wr1, br1)
    ee1 = _edge_xform(edge_attr, We1)

    part1 = _sc_edge_pass(xl1, xr1, ee1, edge_attr, src, dst,
                          _bfr(att1), True)

    ee2 = _edge_xform(edge_attr, We2)

    xl2, xr2, lee2 = _fin1(part1, xl1, xr1, att1, bias1, We1,
                           Wl2, bl2, Wr2, br2, We2)

    part2 = _sc_edge_pass(xl2, xr2, ee2, edge_attr, src, dst,
                          _bfr(att2), False)

    node_pred, graph_pred = _fin2(part2, xl2, xr2, lee2, att2, bias2,
                                  Wn, bn, Wg, bg, batch)
    return node_pred, graph_pred
